# baseline (speedup 1.0000x reference)
import functools
import math

import jax
import jax.numpy as jnp
from jax import lax
from jax.experimental import pallas as pl
from jax.experimental.pallas import tpu as pltpu
from jax.experimental.pallas import tpu_sc as plsc

VOCAB = 1000000
EMBED_DIM = 64
HARMONIC_BASES = 16

_B, _S = 1024, 50
_T = _B * _S
_NC, _NS = 2, 16
_NW = _NC * _NS
_TPW = _T // _NW


def _sc_gather(idx, amp_table, phase_table):
    mesh = plsc.VectorSubcoreMesh(core_axis_name="c", subcore_axis_name="s")

    @functools.partial(
        pl.kernel,
        out_type=(
            jax.ShapeDtypeStruct((_T, HARMONIC_BASES), jnp.float32),
            jax.ShapeDtypeStruct((_T, HARMONIC_BASES), jnp.float32),
        ),
        mesh=mesh,
        scratch_types=[
            pltpu.VMEM((_TPW,), jnp.int32),
            pltpu.VMEM((_TPW, HARMONIC_BASES), jnp.float32),
            pltpu.VMEM((_TPW, HARMONIC_BASES), jnp.float32),
            pltpu.SemaphoreType.DMA,
        ],
        compiler_params=pltpu.CompilerParams(use_tc_tiling_on_sc=False),
    )
    def gather_kernel(idx_hbm, amp_hbm, phase_hbm, amp_out, phase_out,
                      idx_v, rows_a, rows_p, sem):
        wid = lax.axis_index("s") * _NC + lax.axis_index("c")
        base = wid * _TPW
        pltpu.sync_copy(idx_hbm.at[pl.ds(base, _TPW)], idx_v)
        cp_a = pltpu.async_copy(amp_hbm.at[idx_v], rows_a, sem)
        cp_p = pltpu.async_copy(phase_hbm.at[idx_v], rows_p, sem)
        cp_a.wait()
        cp_p.wait()
        pltpu.sync_copy(rows_a, amp_out.at[pl.ds(base, _TPW)])
        pltpu.sync_copy(rows_p, phase_out.at[pl.ds(base, _TPW)])

    return gather_kernel(idx, amp_table, phase_table)


_BT = 2048


def _tc_body(amp_ref, phase_ref, theta_ref, out_ref):
    a = amp_ref[...]
    p = phase_ref[...]
    s_basis = jnp.sin(theta_ref[...])
    c_basis = jnp.cos(theta_ref[...])
    w = a * jnp.cos(p)
    z = a * jnp.sin(p)
    out_ref[...] = (
        jnp.dot(w, s_basis, preferred_element_type=jnp.float32)
        + jnp.dot(z, c_basis, preferred_element_type=jnp.float32)
    )


def _tc_synth(amp_rows, phase_rows, theta):
    grid = (_T // _BT,)
    return pl.pallas_call(
        _tc_body,
        grid=grid,
        in_specs=[
            pl.BlockSpec((_BT, HARMONIC_BASES), lambda i: (i, 0)),
            pl.BlockSpec((_BT, HARMONIC_BASES), lambda i: (i, 0)),
            pl.BlockSpec((HARMONIC_BASES, EMBED_DIM), lambda i: (0, 0)),
        ],
        out_specs=pl.BlockSpec((_BT, EMBED_DIM), lambda i: (i, 0)),
        out_shape=jax.ShapeDtypeStruct((_T, EMBED_DIM), jnp.float32),
    )(amp_rows, phase_rows, theta)


def kernel(x, frequency_amplitudes, frequency_phases, frequencies):
    idx = x.reshape(_T).astype(jnp.int32)
    amp_rows, phase_rows = _sc_gather(idx, frequency_amplitudes,
                                      frequency_phases)
    t = jnp.linspace(0.0, 1.0, EMBED_DIM, dtype=jnp.float32)
    theta = (2.0 * math.pi) * frequencies[:, None] * t[None, :]
    out = _tc_synth(amp_rows, phase_rows, theta)
    return out.reshape(_B, _S, EMBED_DIM)
